# baseline (speedup 1.0000x reference)
import functools

import jax
import jax.numpy as jnp
from jax import lax
from jax.experimental import pallas as pl
from jax.experimental.pallas import tpu as pltpu
from jax.experimental.pallas import tpu_sc as plsc

EMBED = 64
BATCH = 4096
HIST = 200

_NC, _NS = 2, 16
_NW = _NC * _NS
_BPW = BATCH // _NW
_C0, _C1 = 104, 96


def _tanh(x):
    e = jnp.exp(2.0 * x)
    return 1.0 - 2.0 / (e + 1.0)


def _sc_body(idx_hbm, tab_hbm, out_hbm, idx_v, buf_v, out_v, sem):
    wid = lax.axis_index("s") * _NC + lax.axis_index("c")
    base = wid * _BPW
    pltpu.sync_copy(idx_hbm.at[pl.ds(base, _BPW)], idx_v)

    def row(g, carry):
        h1 = pltpu.async_copy(
            tab_hbm.at[idx_v.at[g, pl.ds(0, _C0)]], buf_v.at[pl.ds(0, _C0)], sem)
        h2 = pltpu.async_copy(
            tab_hbm.at[idx_v.at[g, pl.ds(_C0, _C1)]], buf_v.at[pl.ds(_C0, _C1)], sem)
        h1.wait()
        h2.wait()

        ninf = jnp.full((16,), -jnp.inf, dtype=jnp.float32)

        def acc(j, a):
            return (
                jnp.maximum(a[0], buf_v[j, pl.ds(0, 16)]),
                jnp.maximum(a[1], buf_v[j, pl.ds(16, 16)]),
                jnp.maximum(a[2], buf_v[j, pl.ds(32, 16)]),
                jnp.maximum(a[3], buf_v[j, pl.ds(48, 16)]),
            )

        a = lax.fori_loop(0, HIST, acc, (ninf, ninf, ninf, ninf), unroll=4)
        out_v[g, pl.ds(0, 16)] = _tanh(a[0])
        out_v[g, pl.ds(16, 16)] = _tanh(a[1])
        out_v[g, pl.ds(32, 16)] = _tanh(a[2])
        out_v[g, pl.ds(48, 16)] = _tanh(a[3])
        return carry

    lax.fori_loop(0, _BPW, row, 0)
    pltpu.sync_copy(out_v, out_hbm.at[pl.ds(base, _BPW)])


_sc_call = functools.partial(
    pl.kernel,
    mesh=plsc.VectorSubcoreMesh(core_axis_name="c", subcore_axis_name="s"),
    compiler_params=pltpu.CompilerParams(use_tc_tiling_on_sc=False),
    out_type=jax.ShapeDtypeStruct((BATCH, EMBED), jnp.float32),
    scratch_types=[
        pltpu.VMEM((_BPW, HIST), jnp.int32),
        pltpu.VMEM((HIST, EMBED), jnp.float32),
        pltpu.VMEM((_BPW, EMBED), jnp.float32),
        pltpu.SemaphoreType.DMA,
    ],
)(_sc_body)


def kernel(input, embedding):
    return _sc_call(input.astype(jnp.int32), embedding)


# speedup vs baseline: 1.1300x; 1.1300x over previous
import functools

import jax
import jax.numpy as jnp
from jax import lax
from jax.experimental import pallas as pl
from jax.experimental.pallas import tpu as pltpu
from jax.experimental.pallas import tpu_sc as plsc

EMBED = 64
BATCH = 4096
HIST = 200

_NC, _NS = 2, 16
_NW = _NC * _NS
_BPW = BATCH // _NW
_C0, _C1 = 104, 96


def _tanh(x):
    e = jnp.exp(2.0 * x)
    return 1.0 - 2.0 / (e + 1.0)


def _sc_body(idx_hbm, tab_hbm, out_hbm, idx_v, buf0, buf1, out_v, sem0, sem1):
    wid = lax.axis_index("s") * _NC + lax.axis_index("c")
    base = wid * _BPW
    pltpu.sync_copy(idx_hbm.at[pl.ds(base, _BPW)], idx_v)

    def fire(g, buf, sem):
        pltpu.async_copy(
            tab_hbm.at[idx_v.at[g, pl.ds(0, _C0)]], buf.at[pl.ds(0, _C0)], sem)
        pltpu.async_copy(
            tab_hbm.at[idx_v.at[g, pl.ds(_C0, _C1)]], buf.at[pl.ds(_C0, _C1)], sem)

    def drain(buf, sem):
        pltpu.make_async_copy(tab_hbm.at[pl.ds(0, HIST)], buf, sem).wait()

    ninf = jnp.full((16,), -jnp.inf, dtype=jnp.float32)

    def compute(g, buf):
        def acc(j, c):
            a0, a1, a2, a3, b0, b1, b2, b3 = c
            r0 = 2 * j
            r1 = 2 * j + 1
            return (
                jnp.maximum(a0, buf[r0, pl.ds(0, 16)]),
                jnp.maximum(a1, buf[r0, pl.ds(16, 16)]),
                jnp.maximum(a2, buf[r0, pl.ds(32, 16)]),
                jnp.maximum(a3, buf[r0, pl.ds(48, 16)]),
                jnp.maximum(b0, buf[r1, pl.ds(0, 16)]),
                jnp.maximum(b1, buf[r1, pl.ds(16, 16)]),
                jnp.maximum(b2, buf[r1, pl.ds(32, 16)]),
                jnp.maximum(b3, buf[r1, pl.ds(48, 16)]),
            )

        a0, a1, a2, a3, b0, b1, b2, b3 = lax.fori_loop(
            0, HIST // 2, acc, (ninf,) * 8, unroll=4)
        out_v[g, pl.ds(0, 16)] = _tanh(jnp.maximum(a0, b0))
        out_v[g, pl.ds(16, 16)] = _tanh(jnp.maximum(a1, b1))
        out_v[g, pl.ds(32, 16)] = _tanh(jnp.maximum(a2, b2))
        out_v[g, pl.ds(48, 16)] = _tanh(jnp.maximum(a3, b3))

    fire(0, buf0, sem0)

    def pair(t, carry):
        fire(2 * t + 1, buf1, sem1)
        drain(buf0, sem0)
        compute(2 * t, buf0)

        @pl.when(t < _BPW // 2 - 1)
        def _():
            fire(2 * t + 2, buf0, sem0)

        drain(buf1, sem1)
        compute(2 * t + 1, buf1)
        return carry

    lax.fori_loop(0, _BPW // 2, pair, 0)
    pltpu.sync_copy(out_v, out_hbm.at[pl.ds(base, _BPW)])


_sc_call = functools.partial(
    pl.kernel,
    mesh=plsc.VectorSubcoreMesh(core_axis_name="c", subcore_axis_name="s"),
    compiler_params=pltpu.CompilerParams(use_tc_tiling_on_sc=False),
    out_type=jax.ShapeDtypeStruct((BATCH, EMBED), jnp.float32),
    scratch_types=[
        pltpu.VMEM((_BPW, HIST), jnp.int32),
        pltpu.VMEM((HIST, EMBED), jnp.float32),
        pltpu.VMEM((HIST, EMBED), jnp.float32),
        pltpu.VMEM((_BPW, EMBED), jnp.float32),
        pltpu.SemaphoreType.DMA,
        pltpu.SemaphoreType.DMA,
    ],
)(_sc_body)


def kernel(input, embedding):
    return _sc_call(input.astype(jnp.int32), embedding)


# speedup vs baseline: 1.1807x; 1.0449x over previous
import functools

import jax
import jax.numpy as jnp
from jax import lax
from jax.experimental import pallas as pl
from jax.experimental.pallas import tpu as pltpu
from jax.experimental.pallas import tpu_sc as plsc

EMBED = 64
BATCH = 4096
HIST = 200

_NC, _NS = 2, 16
_NW = _NC * _NS
_BPW = BATCH // _NW
_C0, _C1 = 104, 96


def _tanh(x):
    e = jnp.exp(2.0 * x)
    return 1.0 - 2.0 / (e + 1.0)


_NBUF = 4
_CHUNKS = ((0, 56), (56, 56), (112, 56), (168, 32))


def _sc_body(idx_hbm, tab_hbm, out_hbm, idx_v,
             b0, b1, b2, b3, out_v, s0, s1, s2, s3):
    bufs = (b0, b1, b2, b3)
    sems = (s0, s1, s2, s3)
    wid = lax.axis_index("s") * _NC + lax.axis_index("c")
    base = wid * _BPW
    pltpu.sync_copy(idx_hbm.at[pl.ds(base, _BPW)], idx_v)

    def fire(g, b):
        for off, sz in _CHUNKS:
            pltpu.async_copy(
                tab_hbm.at[idx_v.at[g, pl.ds(off, sz)]],
                bufs[b].at[pl.ds(off, sz)], sems[b])

    def drain(b):
        pltpu.make_async_copy(tab_hbm.at[pl.ds(0, HIST)], bufs[b], sems[b]).wait()

    ninf = jnp.full((16,), -jnp.inf, dtype=jnp.float32)

    def compute(g, buf):
        def acc(j, c):
            a0, a1, a2, a3, b0, b1, b2, b3 = c
            r0 = 2 * j
            r1 = 2 * j + 1
            return (
                jnp.maximum(a0, buf[r0, pl.ds(0, 16)]),
                jnp.maximum(a1, buf[r0, pl.ds(16, 16)]),
                jnp.maximum(a2, buf[r0, pl.ds(32, 16)]),
                jnp.maximum(a3, buf[r0, pl.ds(48, 16)]),
                jnp.maximum(b0, buf[r1, pl.ds(0, 16)]),
                jnp.maximum(b1, buf[r1, pl.ds(16, 16)]),
                jnp.maximum(b2, buf[r1, pl.ds(32, 16)]),
                jnp.maximum(b3, buf[r1, pl.ds(48, 16)]),
            )

        a0, a1, a2, a3, b0, b1, b2, b3 = lax.fori_loop(
            0, HIST // 2, acc, (ninf,) * 8, unroll=4)
        out_v[g, pl.ds(0, 16)] = _tanh(jnp.maximum(a0, b0))
        out_v[g, pl.ds(16, 16)] = _tanh(jnp.maximum(a1, b1))
        out_v[g, pl.ds(32, 16)] = _tanh(jnp.maximum(a2, b2))
        out_v[g, pl.ds(48, 16)] = _tanh(jnp.maximum(a3, b3))

    for b in range(_NBUF - 1):
        fire(b, b)

    def quad(t, carry):
        for b in range(_NBUF):
            g = _NBUF * t + b
            drain(b)
            compute(g, bufs[b])

            @pl.when(g + _NBUF - 1 < _BPW)
            def _():
                fire(g + _NBUF - 1, (b + _NBUF - 1) % _NBUF)

        return carry

    lax.fori_loop(0, _BPW // _NBUF, quad, 0)
    pltpu.sync_copy(out_v, out_hbm.at[pl.ds(base, _BPW)])


_sc_call = functools.partial(
    pl.kernel,
    mesh=plsc.VectorSubcoreMesh(core_axis_name="c", subcore_axis_name="s"),
    compiler_params=pltpu.CompilerParams(use_tc_tiling_on_sc=False),
    out_type=jax.ShapeDtypeStruct((BATCH, EMBED), jnp.float32),
    scratch_types=[
        pltpu.VMEM((_BPW, HIST), jnp.int32),
        pltpu.VMEM((HIST, EMBED), jnp.float32),
        pltpu.VMEM((HIST, EMBED), jnp.float32),
        pltpu.VMEM((HIST, EMBED), jnp.float32),
        pltpu.VMEM((HIST, EMBED), jnp.float32),
        pltpu.VMEM((_BPW, EMBED), jnp.float32),
        pltpu.SemaphoreType.DMA,
        pltpu.SemaphoreType.DMA,
        pltpu.SemaphoreType.DMA,
        pltpu.SemaphoreType.DMA,
    ],
)(_sc_body)


def kernel(input, embedding):
    return _sc_call(input.astype(jnp.int32), embedding)


# speedup vs baseline: 1.1858x; 1.0043x over previous
import functools

import jax
import jax.numpy as jnp
from jax import lax
from jax.experimental import pallas as pl
from jax.experimental.pallas import tpu as pltpu
from jax.experimental.pallas import tpu_sc as plsc

EMBED = 64
BATCH = 4096
HIST = 200

_NC, _NS = 2, 16
_NW = _NC * _NS
_BPW = BATCH // _NW
_C0, _C1 = 104, 96


def _tanh(x):
    e = jnp.exp(2.0 * x)
    return 1.0 - 2.0 / (e + 1.0)


_NBUF = 4
_CHUNKS = ((0, 128), (128, 72))


def _sc_body(idx_hbm, tab_hbm, out_hbm, idx_v,
             b0, b1, b2, b3, out_v, s0, s1, s2, s3):
    bufs = (b0, b1, b2, b3)
    sems = (s0, s1, s2, s3)
    wid = lax.axis_index("s") * _NC + lax.axis_index("c")
    base = wid * _BPW
    pltpu.sync_copy(idx_hbm.at[pl.ds(base, _BPW)], idx_v)

    def fire(g, b):
        for off, sz in _CHUNKS:
            pltpu.async_copy(
                tab_hbm.at[idx_v.at[g, pl.ds(off, sz)]],
                bufs[b].at[pl.ds(off, sz)], sems[b])

    def drain(b):
        pltpu.make_async_copy(tab_hbm.at[pl.ds(0, HIST)], bufs[b], sems[b]).wait()

    ninf = jnp.full((16,), -jnp.inf, dtype=jnp.float32)

    def compute(g, buf):
        def acc(j, c):
            a0, a1, a2, a3, b0, b1, b2, b3 = c
            r0 = 2 * j
            r1 = 2 * j + 1
            return (
                jnp.maximum(a0, buf[r0, pl.ds(0, 16)]),
                jnp.maximum(a1, buf[r0, pl.ds(16, 16)]),
                jnp.maximum(a2, buf[r0, pl.ds(32, 16)]),
                jnp.maximum(a3, buf[r0, pl.ds(48, 16)]),
                jnp.maximum(b0, buf[r1, pl.ds(0, 16)]),
                jnp.maximum(b1, buf[r1, pl.ds(16, 16)]),
                jnp.maximum(b2, buf[r1, pl.ds(32, 16)]),
                jnp.maximum(b3, buf[r1, pl.ds(48, 16)]),
            )

        a0, a1, a2, a3, b0, b1, b2, b3 = lax.fori_loop(
            0, HIST // 2, acc, (ninf,) * 8, unroll=4)
        out_v[g, pl.ds(0, 16)] = _tanh(jnp.maximum(a0, b0))
        out_v[g, pl.ds(16, 16)] = _tanh(jnp.maximum(a1, b1))
        out_v[g, pl.ds(32, 16)] = _tanh(jnp.maximum(a2, b2))
        out_v[g, pl.ds(48, 16)] = _tanh(jnp.maximum(a3, b3))

    for b in range(_NBUF - 1):
        fire(b, b)

    def quad(t, carry):
        for b in range(_NBUF):
            g = _NBUF * t + b
            drain(b)
            compute(g, bufs[b])

            @pl.when(g + _NBUF - 1 < _BPW)
            def _():
                fire(g + _NBUF - 1, (b + _NBUF - 1) % _NBUF)

        return carry

    lax.fori_loop(0, _BPW // _NBUF, quad, 0)
    pltpu.sync_copy(out_v, out_hbm.at[pl.ds(base, _BPW)])


_sc_call = functools.partial(
    pl.kernel,
    mesh=plsc.VectorSubcoreMesh(core_axis_name="c", subcore_axis_name="s"),
    compiler_params=pltpu.CompilerParams(use_tc_tiling_on_sc=False),
    out_type=jax.ShapeDtypeStruct((BATCH, EMBED), jnp.float32),
    scratch_types=[
        pltpu.VMEM((_BPW, HIST), jnp.int32),
        pltpu.VMEM((HIST, EMBED), jnp.float32),
        pltpu.VMEM((HIST, EMBED), jnp.float32),
        pltpu.VMEM((HIST, EMBED), jnp.float32),
        pltpu.VMEM((HIST, EMBED), jnp.float32),
        pltpu.VMEM((_BPW, EMBED), jnp.float32),
        pltpu.SemaphoreType.DMA,
        pltpu.SemaphoreType.DMA,
        pltpu.SemaphoreType.DMA,
        pltpu.SemaphoreType.DMA,
    ],
)(_sc_body)


def kernel(input, embedding):
    return _sc_call(input.astype(jnp.int32), embedding)
